# Initial kernel scaffold; baseline (speedup 1.0000x reference)
import jax
import jax.numpy as jnp
from jax.experimental import pallas as pl


def kernel(users, pos_items, neg_items, loss_per_user, w_0, user_embed, item_embed, adj_rows, adj_cols, adj_vals, W, b, noise, drop_mask):
    raise NotImplementedError("write your pallas kernel here")


# speedup vs baseline: 10.5665x; 10.5665x over previous
import functools

import jax
import jax.numpy as jnp
from jax import lax
from jax.experimental import pallas as pl
from jax.experimental.pallas import tpu as pltpu
from jax.experimental.pallas import tpu_sc as plsc

N_USERS = 50000
N_ITEMS = 50000
D = 64
NNZ = 1000000
B = 4096
N_TOT = N_USERS + N_ITEMS
DROP_RATE = 0.5

NC = 2
NS = 16
L = 16
NW = NC * NS
CP = 2 * B
CHUNK = 2048
CHUNKS_PER_TILE = 16
NNZ_PAD = NW * CHUNKS_PER_TILE * CHUNK
EPT = CHUNKS_PER_TILE * CHUNK
MAPW = 50048

_mesh = plsc.VectorSubcoreMesh(core_axis_name="c", subcore_axis_name="s")


def _noise_body(base_ref, noise_ref, out_ref):
    bse = base_ref[...]
    n = noise_ref[...]
    nrm = jnp.sqrt(jnp.sum(n * n, axis=-1, keepdims=True))
    nn = n / jnp.maximum(nrm, 1e-12)
    out_ref[...] = bse + jnp.sign(bse) * nn * 0.1


def _noise_table(base, noise):
    blk = 2000
    grid = N_TOT // blk
    return pl.pallas_call(
        _noise_body,
        grid=(grid,),
        in_specs=[pl.BlockSpec((blk, D), lambda i: (i, 0)),
                  pl.BlockSpec((blk, D), lambda i: (i, 0))],
        out_specs=pl.BlockSpec((blk, D), lambda i: (i, 0)),
        out_shape=jax.ShapeDtypeStruct((N_TOT, D), jnp.float32),
    )(base, noise)


@functools.partial(
    pl.kernel,
    mesh=_mesh,
    compiler_params=pltpu.CompilerParams(needs_layout_passes=False, use_tc_tiling_on_sc=False),
    out_type=[
        jax.ShapeDtypeStruct((NC, CP, D), jnp.float32),
        jax.ShapeDtypeStruct((2, B), jnp.int32),
    ],
    scratch_types=[
        pltpu.VMEM((MAPW,), jnp.int32),
        pltpu.VMEM((B,), jnp.int32),
        pltpu.VMEM((CHUNK,), jnp.int32),
        pltpu.VMEM((CHUNK,), jnp.int32),
        pltpu.VMEM((CHUNK,), jnp.float32),
        pltpu.VMEM((CHUNK,), jnp.float32),
        pltpu.VMEM((CHUNK + 128,), jnp.int32),
        pltpu.VMEM((CHUNK + 128,), jnp.int32),
        pltpu.VMEM((CHUNK + 128,), jnp.float32),
        pltpu.VMEM((L, D), jnp.float32),
        pltpu.VMEM((L, D), jnp.float32),
        pltpu.VMEM_SHARED((CP, D), jnp.float32),
        pltpu.SemaphoreType.DMA,
    ],
)
def _sc_aggregate(neg1_hbm, users_hbm, items_hbm, rows_hbm, cols_hbm,
                  vals_hbm, dmask_hbm, emb_hbm,
                  compact_out, slots_out,
                  mapv, bbuf, rbuf, cbuf, vbuf, dbuf,
                  cslot, ccol, cval, rowbuf, contrib, compact, sem):
    c = lax.axis_index("c")
    s = lax.axis_index("s")
    wid = s * NC + c

    pltpu.sync_copy(neg1_hbm, mapv)
    ramp = lax.iota(jnp.int32, L)

    def scat_pass(parity, node_off, slot_off):
        def body(j, _):
            n = bbuf[pl.ds(j * L, L)] + node_off
            w = n >> 1
            word = plsc.load_gather(mapv, [w])
            slotv = slot_off + j * L + ramp
            if parity == 0:
                neww = (word & jnp.int32(-65536)) | slotv
            else:
                neww = (word & jnp.int32(65535)) | (slotv << 16)
            plsc.store_scatter(mapv, [w], neww, mask=(n & 1) == parity)
            return 0
        lax.fori_loop(0, B // L, body, 0)

    pltpu.sync_copy(users_hbm, bbuf)
    scat_pass(0, 0, 0)
    scat_pass(1, 0, 0)
    pltpu.sync_copy(items_hbm, bbuf)
    scat_pass(0, N_USERS, B)
    scat_pass(1, N_USERS, B)

    zed = jnp.zeros((L,), jnp.float32)
    for i in range(L):
        for dblk in range(D // L):
            contrib[i, pl.ds(dblk * L, L)] = zed
    nstripe = (CP // NS) // L
    def zero_body(t, _):
        pltpu.sync_copy(contrib, compact.at[pl.ds((s * nstripe + t) * L, L)])
        return 0
    lax.fori_loop(0, nstripe, zero_body, 0)
    plsc.subcore_barrier()

    def chunk_body(k, _):
        base = wid * EPT + k * CHUNK
        pltpu.sync_copy(rows_hbm.at[pl.ds(base, CHUNK)], rbuf)
        pltpu.sync_copy(cols_hbm.at[pl.ds(base, CHUNK)], cbuf)
        pltpu.sync_copy(vals_hbm.at[pl.ds(base, CHUNK)], vbuf)
        pltpu.sync_copy(dmask_hbm.at[pl.ds(base, CHUNK)], dbuf)

        def vec_body(j, cnt):
            r = rbuf[pl.ds(j * L, L)]
            cc = cbuf[pl.ds(j * L, L)]
            v = vbuf[pl.ds(j * L, L)] * dbuf[pl.ds(j * L, L)] * (
                1.0 / (1.0 - DROP_RATE))
            word = plsc.load_gather(mapv, [r >> 1])
            slot = (word << ((1 - (r & 1)) * 16)) >> 16
            keep = (slot >= 0) & (v != 0.0)
            plsc.store_compressed(cslot.at[pl.ds(cnt, L)], slot, mask=keep)
            plsc.store_compressed(ccol.at[pl.ds(cnt, L)], cc, mask=keep)
            plsc.store_compressed(cval.at[pl.ds(cnt, L)], v, mask=keep)
            return cnt + jnp.sum(keep.astype(jnp.int32))

        cnt = lax.fori_loop(0, CHUNK // L, vec_body, 0)
        cslot[pl.ds(cnt, L)] = jnp.zeros((L,), jnp.int32)
        ccol[pl.ds(cnt, L)] = jnp.zeros((L,), jnp.int32)
        cval[pl.ds(cnt, L)] = jnp.zeros((L,), jnp.float32)
        ngroups = lax.div(cnt + (L - 1), L)

        def grp_body(g, _):
            colv = ccol[pl.ds(g * L, L)]
            pltpu.async_copy(emb_hbm.at[colv], rowbuf, sem).wait()
            vvec = cval[pl.ds(g * L, L)]
            for i in range(L):
                vv = vvec[i]
                for dblk in range(D // L):
                    contrib[i, pl.ds(dblk * L, L)] = (
                        rowbuf[i, pl.ds(dblk * L, L)] * vv)
            slotv = cslot[pl.ds(g * L, L)]
            pltpu.sync_copy(contrib, compact.at[slotv], add=True)
            return 0

        lax.fori_loop(0, ngroups, grp_body, 0)
        return 0

    lax.fori_loop(0, CHUNKS_PER_TILE, chunk_body, 0)
    plsc.subcore_barrier()

    stripe = CP // NS
    pltpu.sync_copy(compact.at[pl.ds(s * stripe, stripe)],
                    compact_out.at[c, pl.ds(s * stripe, stripe)])

    nb = B // NS

    @pl.when(c == 0)
    def _():
        pltpu.sync_copy(users_hbm.at[pl.ds(s * nb, nb)], rbuf.at[pl.ds(0, nb)])
        def sl_u(t, _):
            u = rbuf[pl.ds(t * L, L)]
            word = plsc.load_gather(mapv, [u >> 1])
            cslot[pl.ds(t * L, L)] = (word << ((1 - (u & 1)) * 16)) >> 16
            return 0
        lax.fori_loop(0, nb // L, sl_u, 0)
        pltpu.sync_copy(cslot.at[pl.ds(0, nb)],
                        slots_out.at[0, pl.ds(s * nb, nb)])

    @pl.when(c == 1)
    def _():
        pltpu.sync_copy(items_hbm.at[pl.ds(s * nb, nb)], rbuf.at[pl.ds(0, nb)])
        def sl_i(t, _):
            it = rbuf[pl.ds(t * L, L)] + N_USERS
            word = plsc.load_gather(mapv, [it >> 1])
            cslot[pl.ds(t * L, L)] = (word << ((1 - (it & 1)) * 16)) >> 16
            return 0
        lax.fori_loop(0, nb // L, sl_i, 0)
        pltpu.sync_copy(cslot.at[pl.ds(0, nb)],
                        slots_out.at[1, pl.ds(s * nb, nb)])


NB2 = B // NW


@functools.partial(
    pl.kernel,
    mesh=_mesh,
    compiler_params=pltpu.CompilerParams(needs_layout_passes=False, use_tc_tiling_on_sc=False),
    out_type=[
        jax.ShapeDtypeStruct((2, B, D), jnp.float32),
        jax.ShapeDtypeStruct((2, B, D), jnp.float32),
        jax.ShapeDtypeStruct((B, D), jnp.float32),
        jax.ShapeDtypeStruct((B, D), jnp.float32),
    ],
    scratch_types=[
        pltpu.VMEM((NB2,), jnp.int32),
        pltpu.VMEM((NB2,), jnp.int32),
        pltpu.VMEM((NB2, D), jnp.float32),
        pltpu.SemaphoreType.DMA,
    ],
)
def _sc_gather(compact_hbm, slots_hbm, users_hbm, items_hbm,
               uemb_hbm, iemb_hbm,
               ut_out, it_out, uon_out, ion_out,
               idxb, idx2, gbuf, sem):
    c = lax.axis_index("c")
    s = lax.axis_index("s")
    wid = s * NC + c
    base = wid * NB2

    def bump(off):
        def body(t, _):
            idx2[pl.ds(t * L, L)] = idxb[pl.ds(t * L, L)] + off
            return 0
        lax.fori_loop(0, NB2 // L, body, 0)

    pltpu.sync_copy(slots_hbm.at[0, pl.ds(base, NB2)], idxb)
    for p in range(NC):
        bump(p * CP)
        pltpu.async_copy(compact_hbm.at[idx2], gbuf, sem).wait()
        pltpu.sync_copy(gbuf, ut_out.at[p, pl.ds(base, NB2)])

    pltpu.sync_copy(slots_hbm.at[1, pl.ds(base, NB2)], idxb)
    for p in range(NC):
        bump(p * CP)
        pltpu.async_copy(compact_hbm.at[idx2], gbuf, sem).wait()
        pltpu.sync_copy(gbuf, it_out.at[p, pl.ds(base, NB2)])

    pltpu.sync_copy(users_hbm.at[pl.ds(base, NB2)], idxb)
    pltpu.async_copy(uemb_hbm.at[idxb], gbuf, sem).wait()
    pltpu.sync_copy(gbuf, uon_out.at[pl.ds(base, NB2)])

    pltpu.sync_copy(items_hbm.at[pl.ds(base, NB2)], idxb)
    pltpu.async_copy(iemb_hbm.at[idxb], gbuf, sem).wait()
    pltpu.sync_copy(gbuf, ion_out.at[pl.ds(base, NB2)])


def _mm_body(x_ref, w_ref, b_ref, o_ref):
    o_ref[...] = lax.dot_general(
        x_ref[...], w_ref[...], (((1,), (1,)), ((), ())),
        preferred_element_type=jnp.float32) + b_ref[...]


def _predictor(x, W, b):
    return pl.pallas_call(
        _mm_body,
        out_shape=jax.ShapeDtypeStruct((2 * B, D), jnp.float32),
    )(x, W, b.reshape(1, D))


def _loss_body(p2u_ref, p2i_ref, utp_ref, itp_ref, o_ref):
    u_t = utp_ref[0] + utp_ref[1]
    i_t = itp_ref[0] + itp_ref[1]
    p2u = p2u_ref[...]
    p2i = p2i_ref[...]

    def cos_loss(p2, z):
        num = jnp.sum(p2 * z, axis=-1)
        den = jnp.maximum(
            jnp.sqrt(jnp.sum(p2 * p2, axis=-1)) *
            jnp.sqrt(jnp.sum(z * z, axis=-1)), 1e-8)
        return -jnp.mean(num / den)

    o_ref[0, 0] = cos_loss(p2u, i_t) / 2.0 + cos_loss(p2i, u_t) / 2.0


def _loss(p2u, p2i, utp, itp):
    return pl.pallas_call(
        _loss_body,
        out_specs=pl.BlockSpec(memory_space=pltpu.SMEM),
        out_shape=jax.ShapeDtypeStruct((1, 1), jnp.float32),
    )(p2u, p2i, utp, itp)


def kernel(users, pos_items, neg_items, loss_per_user, w_0,
           user_embed, item_embed, adj_rows, adj_cols, adj_vals,
           W, b, noise, drop_mask):
    users = users.astype(jnp.int32)
    pos_items = pos_items.astype(jnp.int32)

    base = jnp.concatenate([user_embed, item_embed], axis=0)
    base = jnp.reshape(base, (D, N_TOT)).T
    table = _noise_table(base, noise)

    pad = NNZ_PAD - NNZ
    rows_p = jnp.concatenate([adj_rows.astype(jnp.int32),
                              jnp.zeros((pad,), jnp.int32)])
    cols_p = jnp.concatenate([adj_cols.astype(jnp.int32),
                              jnp.zeros((pad,), jnp.int32)])
    vals_p = jnp.concatenate([adj_vals, jnp.zeros((pad,), jnp.float32)])
    dmask_p = jnp.concatenate([drop_mask, jnp.zeros((pad,), jnp.float32)])
    neg1 = jnp.full((MAPW,), -1, jnp.int32)

    compact, slots = _sc_aggregate(neg1, users, pos_items, rows_p, cols_p,
                                   vals_p, dmask_p, table)
    compact_flat = jnp.reshape(compact, (NC * CP, D))

    utp, itp, u_online, i_online = _sc_gather(
        compact_flat, slots, users, pos_items, user_embed, item_embed)

    x = jnp.concatenate([u_online, i_online], axis=0)
    out = _predictor(x, W, b)
    u1 = out[:B]
    i1 = out[B:]
    p2u = jnp.reshape(u1, (D, B)).T
    p2i = jnp.reshape(i1, (D, B)).T

    loss = _loss(p2u, p2i, utp, itp)
    return jnp.reshape(loss, ())


# speedup vs baseline: 11.3796x; 1.0769x over previous
import functools

import jax
import jax.numpy as jnp
from jax import lax
from jax.experimental import pallas as pl
from jax.experimental.pallas import tpu as pltpu
from jax.experimental.pallas import tpu_sc as plsc

N_USERS = 50000
N_ITEMS = 50000
D = 64
NNZ = 1000000
B = 4096
N_TOT = N_USERS + N_ITEMS
DROP_RATE = 0.5

NC = 2
NS = 16
L = 16
NW = NC * NS
CP = 2 * B
CHUNK = 2048
EPT = 31248
NFULL = 15
TAIL = EPT - NFULL * CHUNK
EXTRA = NNZ - NW * EPT
MAPW = 50048

_mesh = plsc.VectorSubcoreMesh(core_axis_name="c", subcore_axis_name="s")


def _noise_body(u2_ref, i2_ref, noise_ref, out_ref):
    bse = jnp.concatenate([u2_ref[...], i2_ref[...]], axis=1)
    n = noise_ref[...]
    nrm = jnp.sqrt(jnp.sum(n * n, axis=-1, keepdims=True))
    nn = n / jnp.maximum(nrm, 1e-12)
    out_ref[...] = bse + jnp.sign(bse) * nn * 0.1


def _noise_table(u2, i2, noise):
    blk = 2000
    grid = N_TOT // blk
    return pl.pallas_call(
        _noise_body,
        grid=(grid,),
        in_specs=[pl.BlockSpec((blk, D // 2), lambda i: (i, 0)),
                  pl.BlockSpec((blk, D // 2), lambda i: (i, 0)),
                  pl.BlockSpec((blk, D), lambda i: (i, 0))],
        out_specs=pl.BlockSpec((blk, D), lambda i: (i, 0)),
        out_shape=jax.ShapeDtypeStruct((N_TOT, D), jnp.float32),
    )(u2, i2, noise)


@functools.partial(
    pl.kernel,
    mesh=_mesh,
    compiler_params=pltpu.CompilerParams(needs_layout_passes=False, use_tc_tiling_on_sc=False),
    out_type=[
        jax.ShapeDtypeStruct((NC, CP, D), jnp.float32),
        jax.ShapeDtypeStruct((2, B), jnp.int32),
    ],
    scratch_types=[
        pltpu.VMEM((MAPW,), jnp.int32),
        pltpu.VMEM((B,), jnp.int32),
        pltpu.VMEM((CHUNK,), jnp.int32),
        pltpu.VMEM((CHUNK,), jnp.int32),
        pltpu.VMEM((CHUNK,), jnp.float32),
        pltpu.VMEM((CHUNK,), jnp.float32),
        pltpu.VMEM((CHUNK + 128,), jnp.int32),
        pltpu.VMEM((CHUNK + 128,), jnp.int32),
        pltpu.VMEM((CHUNK + 128,), jnp.float32),
        pltpu.VMEM((L, D), jnp.float32),
        pltpu.VMEM((L, D), jnp.float32),
        pltpu.VMEM_SHARED((CP, D), jnp.float32),
        pltpu.SemaphoreType.DMA,
    ],
)
def _sc_aggregate(neg1_hbm, users_hbm, items_hbm, rows_hbm, cols_hbm,
                  vals_hbm, dmask_hbm, emb_hbm,
                  compact_out, slots_out,
                  mapv, bbuf, rbuf, cbuf, vbuf, dbuf,
                  cslot, ccol, cval, rowbuf, contrib, compact, sem):
    c = lax.axis_index("c")
    s = lax.axis_index("s")
    wid = s * NC + c

    pltpu.sync_copy(neg1_hbm, mapv)
    ramp = lax.iota(jnp.int32, L)

    def scat_pass(parity, node_off, slot_off):
        def body(j, _):
            n = bbuf[pl.ds(j * L, L)] + node_off
            w = n >> 1
            word = plsc.load_gather(mapv, [w])
            slotv = slot_off + j * L + ramp
            if parity == 0:
                neww = (word & jnp.int32(-65536)) | slotv
            else:
                neww = (word & jnp.int32(65535)) | (slotv << 16)
            plsc.store_scatter(mapv, [w], neww, mask=(n & 1) == parity)
            return 0
        lax.fori_loop(0, B // L, body, 0)

    pltpu.sync_copy(users_hbm, bbuf)
    scat_pass(0, 0, 0)
    scat_pass(1, 0, 0)
    pltpu.sync_copy(items_hbm, bbuf)
    scat_pass(0, N_USERS, B)
    scat_pass(1, N_USERS, B)

    zed = jnp.zeros((L,), jnp.float32)
    for i in range(L):
        for dblk in range(D // L):
            contrib[i, pl.ds(dblk * L, L)] = zed
    nstripe = (CP // NS) // L
    def zero_body(t, _):
        pltpu.sync_copy(contrib, compact.at[pl.ds((s * nstripe + t) * L, L)])
        return 0
    lax.fori_loop(0, nstripe, zero_body, 0)
    plsc.subcore_barrier()

    def process_span(base, n):
        pltpu.sync_copy(rows_hbm.at[pl.ds(base, n)], rbuf.at[pl.ds(0, n)])
        pltpu.sync_copy(cols_hbm.at[pl.ds(base, n)], cbuf.at[pl.ds(0, n)])
        pltpu.sync_copy(vals_hbm.at[pl.ds(base, n)], vbuf.at[pl.ds(0, n)])
        pltpu.sync_copy(dmask_hbm.at[pl.ds(base, n)], dbuf.at[pl.ds(0, n)])

        def vec_body(j, cnt):
            r = rbuf[pl.ds(j * L, L)]
            cc = cbuf[pl.ds(j * L, L)]
            v = vbuf[pl.ds(j * L, L)] * dbuf[pl.ds(j * L, L)] * (
                1.0 / (1.0 - DROP_RATE))
            word = plsc.load_gather(mapv, [r >> 1])
            slot = (word << ((1 - (r & 1)) * 16)) >> 16
            keep = (slot >= 0) & (v != 0.0)
            plsc.store_compressed(cslot.at[pl.ds(cnt, L)], slot, mask=keep)
            plsc.store_compressed(ccol.at[pl.ds(cnt, L)], cc, mask=keep)
            plsc.store_compressed(cval.at[pl.ds(cnt, L)], v, mask=keep)
            return cnt + jnp.sum(keep.astype(jnp.int32))

        cnt = lax.fori_loop(0, n // L, vec_body, 0)
        cslot[pl.ds(cnt, L)] = jnp.zeros((L,), jnp.int32)
        ccol[pl.ds(cnt, L)] = jnp.zeros((L,), jnp.int32)
        cval[pl.ds(cnt, L)] = jnp.zeros((L,), jnp.float32)
        ngroups = lax.div(cnt + (L - 1), L)

        def grp_body(g, _):
            colv = ccol[pl.ds(g * L, L)]
            pltpu.async_copy(emb_hbm.at[colv], rowbuf, sem).wait()
            vvec = cval[pl.ds(g * L, L)]
            for i in range(L):
                vv = vvec[i]
                for dblk in range(D // L):
                    contrib[i, pl.ds(dblk * L, L)] = (
                        rowbuf[i, pl.ds(dblk * L, L)] * vv)
            slotv = cslot[pl.ds(g * L, L)]
            pltpu.sync_copy(contrib, compact.at[slotv], add=True)
            return 0

        lax.fori_loop(0, ngroups, grp_body, 0)

    def chunk_body(k, _):
        process_span(wid * EPT + k * CHUNK, CHUNK)
        return 0

    lax.fori_loop(0, NFULL, chunk_body, 0)
    process_span(wid * EPT + NFULL * CHUNK, TAIL)

    @pl.when(wid == 0)
    def _():
        process_span(NW * EPT, EXTRA)

    plsc.subcore_barrier()

    stripe = CP // NS
    pltpu.sync_copy(compact.at[pl.ds(s * stripe, stripe)],
                    compact_out.at[c, pl.ds(s * stripe, stripe)])

    nb = B // NS

    @pl.when(c == 0)
    def _():
        pltpu.sync_copy(users_hbm.at[pl.ds(s * nb, nb)], rbuf.at[pl.ds(0, nb)])
        def sl_u(t, _):
            u = rbuf[pl.ds(t * L, L)]
            word = plsc.load_gather(mapv, [u >> 1])
            cslot[pl.ds(t * L, L)] = (word << ((1 - (u & 1)) * 16)) >> 16
            return 0
        lax.fori_loop(0, nb // L, sl_u, 0)
        pltpu.sync_copy(cslot.at[pl.ds(0, nb)],
                        slots_out.at[0, pl.ds(s * nb, nb)])

    @pl.when(c == 1)
    def _():
        pltpu.sync_copy(items_hbm.at[pl.ds(s * nb, nb)], rbuf.at[pl.ds(0, nb)])
        def sl_i(t, _):
            it = rbuf[pl.ds(t * L, L)] + N_USERS
            word = plsc.load_gather(mapv, [it >> 1])
            cslot[pl.ds(t * L, L)] = (word << ((1 - (it & 1)) * 16)) >> 16
            return 0
        lax.fori_loop(0, nb // L, sl_i, 0)
        pltpu.sync_copy(cslot.at[pl.ds(0, nb)],
                        slots_out.at[1, pl.ds(s * nb, nb)])


NB2 = B // NW


@functools.partial(
    pl.kernel,
    mesh=_mesh,
    compiler_params=pltpu.CompilerParams(needs_layout_passes=False, use_tc_tiling_on_sc=False),
    out_type=[
        jax.ShapeDtypeStruct((2, B, D), jnp.float32),
        jax.ShapeDtypeStruct((2, B, D), jnp.float32),
        jax.ShapeDtypeStruct((B, D), jnp.float32),
        jax.ShapeDtypeStruct((B, D), jnp.float32),
    ],
    scratch_types=[
        pltpu.VMEM((NB2,), jnp.int32),
        pltpu.VMEM((NB2,), jnp.int32),
        pltpu.VMEM((NB2, D), jnp.float32),
        pltpu.SemaphoreType.DMA,
    ],
)
def _sc_gather(compact_hbm, slots_hbm, users_hbm, items_hbm,
               uemb_hbm, iemb_hbm,
               ut_out, it_out, uon_out, ion_out,
               idxb, idx2, gbuf, sem):
    c = lax.axis_index("c")
    s = lax.axis_index("s")
    wid = s * NC + c
    base = wid * NB2

    def bump(off):
        def body(t, _):
            idx2[pl.ds(t * L, L)] = idxb[pl.ds(t * L, L)] + off
            return 0
        lax.fori_loop(0, NB2 // L, body, 0)

    pltpu.sync_copy(slots_hbm.at[0, pl.ds(base, NB2)], idxb)
    for p in range(NC):
        bump(p * CP)
        pltpu.async_copy(compact_hbm.at[idx2], gbuf, sem).wait()
        pltpu.sync_copy(gbuf, ut_out.at[p, pl.ds(base, NB2)])

    pltpu.sync_copy(slots_hbm.at[1, pl.ds(base, NB2)], idxb)
    for p in range(NC):
        bump(p * CP)
        pltpu.async_copy(compact_hbm.at[idx2], gbuf, sem).wait()
        pltpu.sync_copy(gbuf, it_out.at[p, pl.ds(base, NB2)])

    pltpu.sync_copy(users_hbm.at[pl.ds(base, NB2)], idxb)
    pltpu.async_copy(uemb_hbm.at[idxb], gbuf, sem).wait()
    pltpu.sync_copy(gbuf, uon_out.at[pl.ds(base, NB2)])

    pltpu.sync_copy(items_hbm.at[pl.ds(base, NB2)], idxb)
    pltpu.async_copy(iemb_hbm.at[idxb], gbuf, sem).wait()
    pltpu.sync_copy(gbuf, ion_out.at[pl.ds(base, NB2)])


def _mm_body(x_ref, w_ref, b_ref, o_ref):
    o_ref[...] = lax.dot_general(
        x_ref[...], w_ref[...], (((1,), (1,)), ((), ())),
        preferred_element_type=jnp.float32) + b_ref[...]


def _predictor(x, W, b):
    return pl.pallas_call(
        _mm_body,
        out_shape=jax.ShapeDtypeStruct((2 * B, D), jnp.float32),
    )(x, W, b.reshape(1, D))


def _loss_body(p2u_ref, p2i_ref, utp_ref, itp_ref, o_ref):
    u_t = utp_ref[0] + utp_ref[1]
    i_t = itp_ref[0] + itp_ref[1]
    p2u = p2u_ref[...]
    p2i = p2i_ref[...]

    def cos_loss(p2, z):
        num = jnp.sum(p2 * z, axis=-1)
        den = jnp.maximum(
            jnp.sqrt(jnp.sum(p2 * p2, axis=-1)) *
            jnp.sqrt(jnp.sum(z * z, axis=-1)), 1e-8)
        return -jnp.mean(num / den)

    o_ref[0, 0] = cos_loss(p2u, i_t) / 2.0 + cos_loss(p2i, u_t) / 2.0


def _loss(p2u, p2i, utp, itp):
    return pl.pallas_call(
        _loss_body,
        out_specs=pl.BlockSpec(memory_space=pltpu.SMEM),
        out_shape=jax.ShapeDtypeStruct((1, 1), jnp.float32),
    )(p2u, p2i, utp, itp)


def kernel(users, pos_items, neg_items, loss_per_user, w_0,
           user_embed, item_embed, adj_rows, adj_cols, adj_vals,
           W, b, noise, drop_mask):
    users = users.astype(jnp.int32)
    pos_items = pos_items.astype(jnp.int32)

    u2 = jnp.reshape(user_embed, (D // 2, N_TOT)).T
    i2 = jnp.reshape(item_embed, (D // 2, N_TOT)).T
    table = _noise_table(u2, i2, noise)

    neg1 = jnp.full((MAPW,), -1, jnp.int32)

    compact, slots = _sc_aggregate(neg1, users, pos_items,
                                   adj_rows.astype(jnp.int32),
                                   adj_cols.astype(jnp.int32),
                                   adj_vals, drop_mask, table)
    compact_flat = jnp.reshape(compact, (NC * CP, D))

    utp, itp, u_online, i_online = _sc_gather(
        compact_flat, slots, users, pos_items, user_embed, item_embed)

    x = jnp.concatenate([u_online, i_online], axis=0)
    out = _predictor(x, W, b)
    u1 = out[:B]
    i1 = out[B:]
    p2u = jnp.reshape(u1, (D, B)).T
    p2i = jnp.reshape(i1, (D, B)).T

    loss = _loss(p2u, p2i, utp, itp)
    return jnp.reshape(loss, ())


# speedup vs baseline: 11.9530x; 1.0504x over previous
import functools

import jax
import jax.numpy as jnp
from jax import lax
from jax.experimental import pallas as pl
from jax.experimental.pallas import tpu as pltpu
from jax.experimental.pallas import tpu_sc as plsc

N_USERS = 50000
N_ITEMS = 50000
D = 64
NNZ = 1000000
B = 4096
N_TOT = N_USERS + N_ITEMS
DROP_RATE = 0.5

NC = 2
NS = 16
L = 16
NW = NC * NS
CP = 2 * B
CHUNK = 2048
EPT = 31248
NFULL = 15
TAIL = EPT - NFULL * CHUNK
EXTRA = NNZ - NW * EPT
MAPW = 50048

_mesh = plsc.VectorSubcoreMesh(core_axis_name="c", subcore_axis_name="s")


def _noise_body(u2_ref, i2_ref, noise_ref, out_ref):
    bse = jnp.concatenate([u2_ref[...], i2_ref[...]], axis=1)
    n = noise_ref[...]
    nrm = jnp.sqrt(jnp.sum(n * n, axis=-1, keepdims=True))
    nn = n / jnp.maximum(nrm, 1e-12)
    out_ref[...] = bse + jnp.sign(bse) * nn * 0.1


def _noise_table(u2, i2, noise):
    blk = 2000
    grid = N_TOT // blk
    return pl.pallas_call(
        _noise_body,
        grid=(grid,),
        in_specs=[pl.BlockSpec((blk, D // 2), lambda i: (i, 0)),
                  pl.BlockSpec((blk, D // 2), lambda i: (i, 0)),
                  pl.BlockSpec((blk, D), lambda i: (i, 0))],
        out_specs=pl.BlockSpec((blk, D), lambda i: (i, 0)),
        out_shape=jax.ShapeDtypeStruct((N_TOT, D), jnp.float32),
    )(u2, i2, noise)


@functools.partial(
    pl.kernel,
    mesh=_mesh,
    compiler_params=pltpu.CompilerParams(needs_layout_passes=False, use_tc_tiling_on_sc=False),
    out_type=[
        jax.ShapeDtypeStruct((NC, CP, D), jnp.float32),
        jax.ShapeDtypeStruct((2, B), jnp.int32),
    ],
    scratch_types=[
        pltpu.VMEM((MAPW,), jnp.int32),
        pltpu.VMEM((B,), jnp.int32),
        pltpu.VMEM((CHUNK,), jnp.int32),
        pltpu.VMEM((CHUNK,), jnp.int32),
        pltpu.VMEM((CHUNK,), jnp.float32),
        pltpu.VMEM((CHUNK,), jnp.float32),
        pltpu.VMEM((CHUNK + 128,), jnp.int32),
        pltpu.VMEM((CHUNK + 128,), jnp.int32),
        pltpu.VMEM((CHUNK + 128,), jnp.float32),
        pltpu.VMEM((L, D), jnp.float32),
        pltpu.VMEM((L, D), jnp.float32),
        pltpu.VMEM_SHARED((CP, D), jnp.float32),
        pltpu.SemaphoreType.DMA,
    ],
)
def _sc_aggregate(neg1_hbm, users_hbm, items_hbm, rows_hbm, cols_hbm,
                  vals_hbm, dmask_hbm, emb_hbm,
                  compact_out, slots_out,
                  mapv, bbuf, rbuf, cbuf, vbuf, dbuf,
                  cslot, ccol, cval, rowbuf, contrib, compact, sem):
    c = lax.axis_index("c")
    s = lax.axis_index("s")
    wid = s * NC + c

    pltpu.sync_copy(neg1_hbm, mapv)
    ramp = lax.iota(jnp.int32, L)

    def scat_pass(parity, node_off, slot_off):
        def body(j, _):
            n = bbuf[pl.ds(j * L, L)] + node_off
            w = n >> 1
            word = plsc.load_gather(mapv, [w])
            slotv = slot_off + j * L + ramp
            if parity == 0:
                neww = (word & jnp.int32(-65536)) | slotv
            else:
                neww = (word & jnp.int32(65535)) | (slotv << 16)
            plsc.store_scatter(mapv, [w], neww, mask=(n & 1) == parity)
            return 0
        lax.fori_loop(0, B // L, body, 0)

    pltpu.sync_copy(users_hbm, bbuf)
    scat_pass(0, 0, 0)
    scat_pass(1, 0, 0)
    pltpu.sync_copy(items_hbm, bbuf)
    scat_pass(0, N_USERS, B)
    scat_pass(1, N_USERS, B)

    zed = jnp.zeros((L,), jnp.float32)
    for i in range(L):
        for dblk in range(D // L):
            contrib[i, pl.ds(dblk * L, L)] = zed
    nstripe = (CP // NS) // L
    def zero_body(t, _):
        pltpu.sync_copy(contrib, compact.at[pl.ds((s * nstripe + t) * L, L)])
        return 0
    lax.fori_loop(0, nstripe, zero_body, 0)
    plsc.subcore_barrier()

    def process_span(base, n):
        h1 = pltpu.async_copy(rows_hbm.at[pl.ds(base, n)], rbuf.at[pl.ds(0, n)], sem)
        h2 = pltpu.async_copy(cols_hbm.at[pl.ds(base, n)], cbuf.at[pl.ds(0, n)], sem)
        h3 = pltpu.async_copy(vals_hbm.at[pl.ds(base, n)], vbuf.at[pl.ds(0, n)], sem)
        h4 = pltpu.async_copy(dmask_hbm.at[pl.ds(base, n)], dbuf.at[pl.ds(0, n)], sem)
        h1.wait(); h2.wait(); h3.wait(); h4.wait()

        def vec_body(j, cnt):
            r = rbuf[pl.ds(j * L, L)]
            cc = cbuf[pl.ds(j * L, L)]
            v = vbuf[pl.ds(j * L, L)] * dbuf[pl.ds(j * L, L)] * (
                1.0 / (1.0 - DROP_RATE))
            word = plsc.load_gather(mapv, [r >> 1])
            slot = (word << ((1 - (r & 1)) * 16)) >> 16
            keep = (slot >= 0) & (v != 0.0)
            plsc.store_compressed(cslot.at[pl.ds(cnt, L)], slot, mask=keep)
            plsc.store_compressed(ccol.at[pl.ds(cnt, L)], cc, mask=keep)
            plsc.store_compressed(cval.at[pl.ds(cnt, L)], v, mask=keep)
            return cnt + jnp.sum(keep.astype(jnp.int32))

        cnt = lax.fori_loop(0, n // L, vec_body, 0)
        cslot[pl.ds(cnt, L)] = jnp.zeros((L,), jnp.int32)
        ccol[pl.ds(cnt, L)] = jnp.zeros((L,), jnp.int32)
        cval[pl.ds(cnt, L)] = jnp.zeros((L,), jnp.float32)
        ngroups = lax.div(cnt + (L - 1), L)

        def grp_body(g, _):
            colv = ccol[pl.ds(g * L, L)]
            pltpu.async_copy(emb_hbm.at[colv], rowbuf, sem).wait()
            vvec = cval[pl.ds(g * L, L)]
            for i in range(L):
                vv = vvec[i]
                for dblk in range(D // L):
                    contrib[i, pl.ds(dblk * L, L)] = (
                        rowbuf[i, pl.ds(dblk * L, L)] * vv)
            slotv = cslot[pl.ds(g * L, L)]
            pltpu.sync_copy(contrib, compact.at[slotv], add=True)
            return 0

        lax.fori_loop(0, ngroups, grp_body, 0)

    def chunk_body(k, _):
        process_span(wid * EPT + k * CHUNK, CHUNK)
        return 0

    lax.fori_loop(0, NFULL, chunk_body, 0)
    process_span(wid * EPT + NFULL * CHUNK, TAIL)

    @pl.when(wid == 0)
    def _():
        process_span(NW * EPT, EXTRA)

    plsc.subcore_barrier()

    stripe = CP // NS
    pltpu.sync_copy(compact.at[pl.ds(s * stripe, stripe)],
                    compact_out.at[c, pl.ds(s * stripe, stripe)])

    nb = B // NS

    @pl.when(c == 0)
    def _():
        pltpu.sync_copy(users_hbm.at[pl.ds(s * nb, nb)], rbuf.at[pl.ds(0, nb)])
        def sl_u(t, _):
            u = rbuf[pl.ds(t * L, L)]
            word = plsc.load_gather(mapv, [u >> 1])
            cslot[pl.ds(t * L, L)] = (word << ((1 - (u & 1)) * 16)) >> 16
            return 0
        lax.fori_loop(0, nb // L, sl_u, 0)
        pltpu.sync_copy(cslot.at[pl.ds(0, nb)],
                        slots_out.at[0, pl.ds(s * nb, nb)])

    @pl.when(c == 1)
    def _():
        pltpu.sync_copy(items_hbm.at[pl.ds(s * nb, nb)], rbuf.at[pl.ds(0, nb)])
        def sl_i(t, _):
            it = rbuf[pl.ds(t * L, L)] + N_USERS
            word = plsc.load_gather(mapv, [it >> 1])
            cslot[pl.ds(t * L, L)] = (word << ((1 - (it & 1)) * 16)) >> 16
            return 0
        lax.fori_loop(0, nb // L, sl_i, 0)
        pltpu.sync_copy(cslot.at[pl.ds(0, nb)],
                        slots_out.at[1, pl.ds(s * nb, nb)])


NB2 = B // NW


@functools.partial(
    pl.kernel,
    mesh=_mesh,
    compiler_params=pltpu.CompilerParams(needs_layout_passes=False, use_tc_tiling_on_sc=False),
    out_type=[
        jax.ShapeDtypeStruct((2, B, D), jnp.float32),
        jax.ShapeDtypeStruct((2, B, D), jnp.float32),
        jax.ShapeDtypeStruct((B, D), jnp.float32),
        jax.ShapeDtypeStruct((B, D), jnp.float32),
    ],
    scratch_types=[
        pltpu.VMEM((NB2,), jnp.int32),
        pltpu.VMEM((NB2,), jnp.int32),
        pltpu.VMEM((NB2, D), jnp.float32),
        pltpu.SemaphoreType.DMA,
    ],
)
def _sc_gather(compact_hbm, slots_hbm, users_hbm, items_hbm,
               uemb_hbm, iemb_hbm,
               ut_out, it_out, uon_out, ion_out,
               idxb, idx2, gbuf, sem):
    c = lax.axis_index("c")
    s = lax.axis_index("s")
    wid = s * NC + c
    base = wid * NB2

    def bump(off):
        def body(t, _):
            idx2[pl.ds(t * L, L)] = idxb[pl.ds(t * L, L)] + off
            return 0
        lax.fori_loop(0, NB2 // L, body, 0)

    pltpu.sync_copy(slots_hbm.at[0, pl.ds(base, NB2)], idxb)
    for p in range(NC):
        bump(p * CP)
        pltpu.async_copy(compact_hbm.at[idx2], gbuf, sem).wait()
        pltpu.sync_copy(gbuf, ut_out.at[p, pl.ds(base, NB2)])

    pltpu.sync_copy(slots_hbm.at[1, pl.ds(base, NB2)], idxb)
    for p in range(NC):
        bump(p * CP)
        pltpu.async_copy(compact_hbm.at[idx2], gbuf, sem).wait()
        pltpu.sync_copy(gbuf, it_out.at[p, pl.ds(base, NB2)])

    pltpu.sync_copy(users_hbm.at[pl.ds(base, NB2)], idxb)
    pltpu.async_copy(uemb_hbm.at[idxb], gbuf, sem).wait()
    pltpu.sync_copy(gbuf, uon_out.at[pl.ds(base, NB2)])

    pltpu.sync_copy(items_hbm.at[pl.ds(base, NB2)], idxb)
    pltpu.async_copy(iemb_hbm.at[idxb], gbuf, sem).wait()
    pltpu.sync_copy(gbuf, ion_out.at[pl.ds(base, NB2)])


def _mm_body(x_ref, w_ref, b_ref, o_ref):
    o_ref[...] = lax.dot_general(
        x_ref[...], w_ref[...], (((1,), (1,)), ((), ())),
        preferred_element_type=jnp.float32) + b_ref[...]


def _predictor(x, W, b):
    return pl.pallas_call(
        _mm_body,
        out_shape=jax.ShapeDtypeStruct((2 * B, D), jnp.float32),
    )(x, W, b.reshape(1, D))


def _loss_body(p2u_ref, p2i_ref, utp_ref, itp_ref, o_ref):
    u_t = utp_ref[0] + utp_ref[1]
    i_t = itp_ref[0] + itp_ref[1]
    p2u = p2u_ref[...]
    p2i = p2i_ref[...]

    def cos_loss(p2, z):
        num = jnp.sum(p2 * z, axis=-1)
        den = jnp.maximum(
            jnp.sqrt(jnp.sum(p2 * p2, axis=-1)) *
            jnp.sqrt(jnp.sum(z * z, axis=-1)), 1e-8)
        return -jnp.mean(num / den)

    o_ref[0, 0] = cos_loss(p2u, i_t) / 2.0 + cos_loss(p2i, u_t) / 2.0


def _loss(p2u, p2i, utp, itp):
    return pl.pallas_call(
        _loss_body,
        out_specs=pl.BlockSpec(memory_space=pltpu.SMEM),
        out_shape=jax.ShapeDtypeStruct((1, 1), jnp.float32),
    )(p2u, p2i, utp, itp)


def kernel(users, pos_items, neg_items, loss_per_user, w_0,
           user_embed, item_embed, adj_rows, adj_cols, adj_vals,
           W, b, noise, drop_mask):
    users = users.astype(jnp.int32)
    pos_items = pos_items.astype(jnp.int32)

    u2 = jnp.reshape(user_embed, (D // 2, N_TOT)).T
    i2 = jnp.reshape(item_embed, (D // 2, N_TOT)).T
    table = _noise_table(u2, i2, noise)

    neg1 = jnp.full((MAPW,), -1, jnp.int32)

    compact, slots = _sc_aggregate(neg1, users, pos_items,
                                   adj_rows.astype(jnp.int32),
                                   adj_cols.astype(jnp.int32),
                                   adj_vals, drop_mask, table)
    compact_flat = jnp.reshape(compact, (NC * CP, D))

    utp, itp, u_online, i_online = _sc_gather(
        compact_flat, slots, users, pos_items, user_embed, item_embed)

    x = jnp.concatenate([u_online, i_online], axis=0)
    out = _predictor(x, W, b)
    u1 = out[:B]
    i1 = out[B:]
    p2u = jnp.reshape(u1, (D, B)).T
    p2i = jnp.reshape(i1, (D, B)).T

    loss = _loss(p2u, p2i, utp, itp)
    return jnp.reshape(loss, ())
